# baseline (speedup 1.0000x reference)
import functools

import jax
import jax.numpy as jnp
from jax import lax
from jax.experimental import pallas as pl
from jax.experimental.pallas import tpu as pltpu
from jax.experimental.pallas import tpu_sc as plsc

B, C, H, W = 4, 192, 224, 224
HW = H * W
L = 16
ROWS_PER_BLK = 8
BLK = ROWS_PER_BLK * W
NBLK = H // ROWS_PER_BLK
NPAIR = NBLK // 2
VPB = BLK // L
VPR = W // L
BH = B // 2
NUNITS = BH * C
NWORKERS = 32
UPW = NUNITS // NWORKERS
PPW = UPW // 2
WPB = NWORKERS // BH
SX = W / (W - 1.0)
SY = H / (H - 1.0)
DIV14_MUL, DIV14_SHIFT = 18725, 18


def _floor_to_int(v):
    t = v.astype(jnp.int32)
    tf = t.astype(jnp.float32)
    i0 = t - jnp.where(tf > v, 1, 0).astype(jnp.int32)
    return i0, i0.astype(jnp.float32)


def _warp_body(img_hbm, flow_hbm, out_hbm, img0buf, img1buf, fbufA, fbufB,
               outbufA, outbufB, xsbuf, sem_i, sem_fA, sem_fB, sem_oA, sem_oB):
    wid = lax.axis_index("s") * 2 + lax.axis_index("c")
    b = wid // WPB

    def init_xs(i, _):
        xv = lax.iota(jnp.int32, L) + i * L
        xsbuf[pl.ds(i * L, L)] = xv.astype(jnp.float32) * SX - 0.5
        return 0
    lax.fori_loop(0, VPR, init_xs, 0)

    def compute_block(blk, fbuf, outbuf):
        @plsc.parallel_loop(0, VPB, unroll=4)
        def _(v):
            q = (v * DIV14_MUL) >> DIV14_SHIFT
            xv = v - q * VPR
            y = blk * ROWS_PER_BLK + q
            lv = v * L
            dxv = fbuf[0, pl.ds(lv, L)]
            dyv = fbuf[1, pl.ds(lv, L)]
            ix = xsbuf[pl.ds(xv * L, L)] + dxv * SX
            ys = y.astype(jnp.float32) * SY - 0.5
            iy = jnp.full((L,), ys, jnp.float32) + dyv * SY

            ix0, ix0f = _floor_to_int(ix)
            iy0, iy0f = _floor_to_int(iy)
            fx = ix - ix0f
            fy = iy - iy0f

            ix0c = jnp.minimum(jnp.maximum(ix0, 0), W - 1)
            ix1c = jnp.minimum(jnp.maximum(ix0 + 1, 0), W - 1)
            iy0c = jnp.minimum(jnp.maximum(iy0, 0), H - 1)
            iy1c = jnp.minimum(jnp.maximum(iy0 + 1, 0), H - 1)

            zero = jnp.zeros((L,), jnp.float32)
            wx0 = jnp.where((ix0f >= 0.0) & (ix0f <= W - 1.0), 1.0 - fx, zero)
            wx1 = jnp.where((ix0f >= -1.0) & (ix0f <= W - 2.0), fx, zero)
            wy0 = jnp.where((iy0f >= 0.0) & (iy0f <= H - 1.0), 1.0 - fy, zero)
            wy1 = jnp.where((iy0f >= -1.0) & (iy0f <= H - 2.0), fy, zero)

            row0 = iy0c * W
            row1 = iy1c * W
            i00 = row0 + ix0c
            i01 = row0 + ix1c
            i10 = row1 + ix0c
            i11 = row1 + ix1c

            a00 = plsc.load_gather(img0buf, [i00])
            a01 = plsc.load_gather(img0buf, [i01])
            a10 = plsc.load_gather(img0buf, [i10])
            a11 = plsc.load_gather(img0buf, [i11])
            acc0 = (a00 * wx0 + a01 * wx1) * wy0 + (a10 * wx0 + a11 * wx1) * wy1
            outbuf[0, pl.ds(lv, L)] = acc0

            b00 = plsc.load_gather(img1buf, [i00])
            b01 = plsc.load_gather(img1buf, [i01])
            b10 = plsc.load_gather(img1buf, [i10])
            b11 = plsc.load_gather(img1buf, [i11])
            acc1 = (b00 * wx0 + b01 * wx1) * wy0 + (b10 * wx0 + b11 * wx1) * wy1
            outbuf[1, pl.ds(lv, L)] = acc1

    def flow_copy(blk, fbuf, sem):
        return pltpu.make_async_copy(
            flow_hbm.at[b, :, pl.ds(blk * BLK, BLK)], fbuf, sem)

    def out_copy(unit2, blk, outbuf, sem):
        return pltpu.make_async_copy(
            outbuf, out_hbm.at[pl.ds(unit2, 2), pl.ds(blk * BLK, BLK)], sem)

    def pair_unit_body(p, _):
        unit2 = wid * UPW + p * 2
        flow_copy(0, fbufA, sem_fA).start()
        pltpu.make_async_copy(img_hbm.at[unit2], img0buf, sem_i).start()
        pltpu.make_async_copy(img_hbm.at[unit2 + 1], img1buf, sem_i).start()
        pltpu.make_async_copy(img_hbm.at[unit2], img0buf, sem_i).wait()
        pltpu.make_async_copy(img_hbm.at[unit2 + 1], img1buf, sem_i).wait()

        def pair_body(gg, _):
            a_blk = gg * 2
            b_blk = gg * 2 + 1
            flow_copy(b_blk, fbufB, sem_fB).start()
            flow_copy(a_blk, fbufA, sem_fA).wait()

            @pl.when(gg > 0)
            def _():
                out_copy(unit2, a_blk - 2, outbufA, sem_oA).wait()
            compute_block(a_blk, fbufA, outbufA)
            out_copy(unit2, a_blk, outbufA, sem_oA).start()

            @pl.when(gg < NPAIR - 1)
            def _():
                flow_copy(a_blk + 2, fbufA, sem_fA).start()
            flow_copy(b_blk, fbufB, sem_fB).wait()

            @pl.when(gg > 0)
            def _():
                out_copy(unit2, b_blk - 2, outbufB, sem_oB).wait()
            compute_block(b_blk, fbufB, outbufB)
            out_copy(unit2, b_blk, outbufB, sem_oB).start()
            return 0
        lax.fori_loop(0, NPAIR, pair_body, 0)
        out_copy(unit2, NBLK - 2, outbufA, sem_oA).wait()
        out_copy(unit2, NBLK - 1, outbufB, sem_oB).wait()
        return 0
    lax.fori_loop(0, PPW, pair_unit_body, 0)


@jax.jit
def _warp(img2, flow):
    mesh = plsc.VectorSubcoreMesh(core_axis_name="c", subcore_axis_name="s")
    f = functools.partial(
        pl.kernel,
        mesh=mesh,
        compiler_params=pltpu.CompilerParams(needs_layout_passes=False),
        out_type=jax.ShapeDtypeStruct((NUNITS, HW), jnp.float32),
        scratch_types=[
            pltpu.VMEM((HW,), jnp.float32),
            pltpu.VMEM((HW,), jnp.float32),
            pltpu.VMEM((2, BLK), jnp.float32),
            pltpu.VMEM((2, BLK), jnp.float32),
            pltpu.VMEM((2, BLK), jnp.float32),
            pltpu.VMEM((2, BLK), jnp.float32),
            pltpu.VMEM((W,), jnp.float32),
            pltpu.SemaphoreType.DMA,
            pltpu.SemaphoreType.DMA,
            pltpu.SemaphoreType.DMA,
            pltpu.SemaphoreType.DMA,
            pltpu.SemaphoreType.DMA,
        ],
    )(_warp_body)
    return f(img2, flow)


def kernel(input_image, delta_x, delta_y):
    flow = jnp.concatenate(
        [delta_x.reshape(B, 1, HW), delta_y.reshape(B, 1, HW)], axis=1)
    in1 = input_image[:BH].reshape(NUNITS, HW)
    in2 = input_image[BH:].reshape(NUNITS, HW)
    o1 = _warp(in1, flow[:BH]).reshape(BH, C, H, W)
    o2 = _warp(in2, flow[BH:]).reshape(BH, C, H, W)
    return jnp.concatenate([o1, o2], axis=0)


# speedup vs baseline: 1.2234x; 1.2234x over previous
import functools

import jax
import jax.numpy as jnp
from jax import lax
from jax.experimental import pallas as pl
from jax.experimental.pallas import tpu as pltpu
from jax.experimental.pallas import tpu_sc as plsc

B, C, H, W = 4, 192, 224, 224
HW = H * W
L = 16
ROWS_PER_BLK = 8
BLK = ROWS_PER_BLK * W
NBLK = H // ROWS_PER_BLK
NPAIR = NBLK // 2
VPB = BLK // L
VPR = W // L
NUNITS = B * C
NWORKERS = 32
UPW = NUNITS // NWORKERS
PPW = UPW // 2
WPB = NWORKERS // B
SX = W / (W - 1.0)
SY = H / (H - 1.0)
DIV14_MUL, DIV14_SHIFT = 18725, 18


def _floor_to_int(v):
    t = v.astype(jnp.int32)
    tf = t.astype(jnp.float32)
    i0 = t - jnp.where(tf > v, 1, 0).astype(jnp.int32)
    return i0, i0.astype(jnp.float32)


def _warp_body(img_hbm, flow_hbm, out_hbm, img0buf, img1buf, fbufA, fbufB,
               outbufA, outbufB, xsbuf, sem_i, sem_fA, sem_fB, sem_oA, sem_oB):
    wid = lax.axis_index("s") * 2 + lax.axis_index("c")
    b = wid // WPB

    def init_xs(i, _):
        xv = lax.iota(jnp.int32, L) + i * L
        xsbuf[pl.ds(i * L, L)] = xv.astype(jnp.float32) * SX - 0.5
        return 0
    lax.fori_loop(0, VPR, init_xs, 0)

    def compute_block(blk, fbuf, outbuf):
        @plsc.parallel_loop(0, VPB, unroll=4)
        def _(v):
            q = (v * DIV14_MUL) >> DIV14_SHIFT
            xv = v - q * VPR
            y = blk * ROWS_PER_BLK + q
            lv = v * L
            dxv = fbuf[0, pl.ds(lv, L)]
            dyv = fbuf[1, pl.ds(lv, L)]
            ix = xsbuf[pl.ds(xv * L, L)] + dxv * SX
            ys = y.astype(jnp.float32) * SY - 0.5
            iy = jnp.full((L,), ys, jnp.float32) + dyv * SY

            ix0, ix0f = _floor_to_int(ix)
            iy0, iy0f = _floor_to_int(iy)
            fx = ix - ix0f
            fy = iy - iy0f

            ix0c = jnp.minimum(jnp.maximum(ix0, 0), W - 1)
            ix1c = jnp.minimum(jnp.maximum(ix0 + 1, 0), W - 1)
            iy0c = jnp.minimum(jnp.maximum(iy0, 0), H - 1)
            iy1c = jnp.minimum(jnp.maximum(iy0 + 1, 0), H - 1)

            zero = jnp.zeros((L,), jnp.float32)
            wx0 = jnp.where((ix0f >= 0.0) & (ix0f <= W - 1.0), 1.0 - fx, zero)
            wx1 = jnp.where((ix0f >= -1.0) & (ix0f <= W - 2.0), fx, zero)
            wy0 = jnp.where((iy0f >= 0.0) & (iy0f <= H - 1.0), 1.0 - fy, zero)
            wy1 = jnp.where((iy0f >= -1.0) & (iy0f <= H - 2.0), fy, zero)

            a00 = plsc.load_gather(img0buf, [iy0c, ix0c])
            a01 = plsc.load_gather(img0buf, [iy0c, ix1c])
            a10 = plsc.load_gather(img0buf, [iy1c, ix0c])
            a11 = plsc.load_gather(img0buf, [iy1c, ix1c])
            acc0 = (a00 * wx0 + a01 * wx1) * wy0 + (a10 * wx0 + a11 * wx1) * wy1
            outbuf[0, q, pl.ds(xv * L, L)] = acc0

            b00 = plsc.load_gather(img1buf, [iy0c, ix0c])
            b01 = plsc.load_gather(img1buf, [iy0c, ix1c])
            b10 = plsc.load_gather(img1buf, [iy1c, ix0c])
            b11 = plsc.load_gather(img1buf, [iy1c, ix1c])
            acc1 = (b00 * wx0 + b01 * wx1) * wy0 + (b10 * wx0 + b11 * wx1) * wy1
            outbuf[1, q, pl.ds(xv * L, L)] = acc1

    def flow_copy(blk, fbuf, sem):
        return pltpu.make_async_copy(
            flow_hbm.at[b, :, pl.ds(blk * BLK, BLK)], fbuf, sem)

    def out_copy(unit2, blk, outbuf, sem):
        r0 = blk * ROWS_PER_BLK
        return pltpu.make_async_copy(
            outbuf, out_hbm.at[pl.ds(unit2, 2), pl.ds(r0, ROWS_PER_BLK), :],
            sem)

    def pair_unit_body(p, _):
        unit2 = wid * UPW + p * 2
        flow_copy(0, fbufA, sem_fA).start()
        pltpu.make_async_copy(img_hbm.at[unit2], img0buf, sem_i).start()
        pltpu.make_async_copy(img_hbm.at[unit2 + 1], img1buf, sem_i).start()
        pltpu.make_async_copy(img_hbm.at[unit2], img0buf, sem_i).wait()
        pltpu.make_async_copy(img_hbm.at[unit2 + 1], img1buf, sem_i).wait()

        def pair_body(gg, _):
            a_blk = gg * 2
            b_blk = gg * 2 + 1
            flow_copy(b_blk, fbufB, sem_fB).start()
            flow_copy(a_blk, fbufA, sem_fA).wait()

            @pl.when(gg > 0)
            def _():
                out_copy(unit2, a_blk - 2, outbufA, sem_oA).wait()
            compute_block(a_blk, fbufA, outbufA)
            out_copy(unit2, a_blk, outbufA, sem_oA).start()

            @pl.when(gg < NPAIR - 1)
            def _():
                flow_copy(a_blk + 2, fbufA, sem_fA).start()
            flow_copy(b_blk, fbufB, sem_fB).wait()

            @pl.when(gg > 0)
            def _():
                out_copy(unit2, b_blk - 2, outbufB, sem_oB).wait()
            compute_block(b_blk, fbufB, outbufB)
            out_copy(unit2, b_blk, outbufB, sem_oB).start()
            return 0
        lax.fori_loop(0, NPAIR, pair_body, 0)
        out_copy(unit2, NBLK - 2, outbufA, sem_oA).wait()
        out_copy(unit2, NBLK - 1, outbufB, sem_oB).wait()
        return 0
    lax.fori_loop(0, PPW, pair_unit_body, 0)


@jax.jit
def _warp(img3, flow):
    mesh = plsc.VectorSubcoreMesh(core_axis_name="c", subcore_axis_name="s")
    f = functools.partial(
        pl.kernel,
        mesh=mesh,
        compiler_params=pltpu.CompilerParams(needs_layout_passes=False),
        out_type=jax.ShapeDtypeStruct((NUNITS, H, W), jnp.float32),
        scratch_types=[
            pltpu.VMEM((H, W), jnp.float32),
            pltpu.VMEM((H, W), jnp.float32),
            pltpu.VMEM((2, BLK), jnp.float32),
            pltpu.VMEM((2, BLK), jnp.float32),
            pltpu.VMEM((2, ROWS_PER_BLK, W), jnp.float32),
            pltpu.VMEM((2, ROWS_PER_BLK, W), jnp.float32),
            pltpu.VMEM((W,), jnp.float32),
            pltpu.SemaphoreType.DMA,
            pltpu.SemaphoreType.DMA,
            pltpu.SemaphoreType.DMA,
            pltpu.SemaphoreType.DMA,
            pltpu.SemaphoreType.DMA,
        ],
    )(_warp_body)
    return f(img3, flow)


def kernel(input_image, delta_x, delta_y):
    flow = jnp.concatenate(
        [delta_x.reshape(B, 1, HW), delta_y.reshape(B, 1, HW)], axis=1)
    img3 = input_image.reshape(NUNITS, H, W)
    out = _warp(img3, flow)
    return out.reshape(B, C, H, W)
